# baseline (speedup 1.0000x reference)
import functools

import jax
import jax.numpy as jnp
from jax import lax
from jax.experimental import pallas as pl
from jax.experimental.pallas import tpu as pltpu
from jax.experimental.pallas import tpu_sc as plsc

_L = 16
_NBUF = 2
_PAIR = 2


def _emb_kernel(B, T, V, D, NC, NS):
    NW = NC * NS
    TCH = T // NW
    NP = B // _PAIR
    mesh = plsc.VectorSubcoreMesh(core_axis_name="c", subcore_axis_name="s")

    @functools.partial(
        pl.kernel,
        mesh=mesh,
        out_type=jax.ShapeDtypeStruct((B * T, D), jnp.float32),
        scratch_types=(
            [pltpu.VMEM((B * TCH,), jnp.int32),
             pltpu.VMEM((TCH, D), jnp.float32)]
            + [pltpu.VMEM((_PAIR * TCH, D), jnp.float32) for _ in range(_NBUF)]
            + [pltpu.SemaphoreType.DMA for _ in range(2 * _NBUF + 1)]
        ),
    )
    def body(idx_hbm, tok_hbm, pos_hbm, out_hbm, idx_v, pos_v, *bufs_sems):
        rows = bufs_sems[:_NBUF]
        gsem = bufs_sems[_NBUF:2 * _NBUF]
        wsem = bufs_sems[2 * _NBUF:3 * _NBUF]
        psem = bufs_sems[3 * _NBUF]
        wid = lax.axis_index("s") * NC + lax.axis_index("c")
        t0 = wid * TCH

        idx_d = [pltpu.async_copy(idx_hbm.at[b, pl.ds(t0, TCH)],
                                  idx_v.at[pl.ds(b * TCH, TCH)], psem)
                 for b in range(B)]
        pos_d = pltpu.async_copy(pos_hbm.at[pl.ds(t0, TCH)], pos_v, psem)
        for d in idx_d:
            d.wait()

        def gather(p):
            return pltpu.async_copy(
                tok_hbm.at[idx_v.at[pl.ds(p * _PAIR * TCH, _PAIR * TCH)]],
                rows[p % _NBUF], gsem[p % _NBUF])

        gd = [None] * _NBUF
        wd = [None] * _NBUF
        gd[0] = gather(0)
        pos_d.wait()

        for p in range(NP):
            cur = p % _NBUF
            if p + 1 < NP:
                nb = (p + 1) % _NBUF
                for d in (wd[nb] or ()):
                    d.wait()
                gd[nb] = gather(p + 1)
            gd[cur].wait()
            rcur = rows[cur]

            def per_row(r, carry, rcur=rcur):
                for j in range(D // _L):
                    sl = pl.ds(j * _L, _L)
                    pv = pos_v[r, sl]
                    for h in range(_PAIR):
                        plsc.addupdate(rcur.at[r + h * TCH, sl], pv)
                return carry

            lax.fori_loop(0, TCH, per_row, 0)
            wd[cur] = [
                pltpu.async_copy(
                    rcur.at[pl.ds(h * TCH, TCH)],
                    out_hbm.at[pl.ds((p * _PAIR + h) * T + t0, TCH)],
                    wsem[cur])
                for h in range(_PAIR)
            ]
        for ds_ in wd:
            for d in (ds_ or ()):
                d.wait()

    return body


def kernel(idx, tok_table, pos_table):
    B, T = idx.shape
    V, D = tok_table.shape
    info = plsc.get_sparse_core_info()
    NC, NS = info.num_cores, info.num_subcores
    fn = _emb_kernel(B, T, V, D, NC, NS)
    out = fn(idx.astype(jnp.int32), tok_table, pos_table)
    return out.reshape(B, T, D)


# speedup vs baseline: 1.0237x; 1.0237x over previous
import functools

import jax
import jax.numpy as jnp
from jax import lax
from jax.experimental import pallas as pl
from jax.experimental.pallas import tpu as pltpu
from jax.experimental.pallas import tpu_sc as plsc

_L = 16
_NBUF = 2
_PAIR = 2


def _emb_kernel(B, T, V, D, NC, NS):
    NW = NC * NS
    TCH = T // NW
    NP = B // _PAIR
    mesh = plsc.VectorSubcoreMesh(core_axis_name="c", subcore_axis_name="s")

    @functools.partial(
        pl.kernel,
        mesh=mesh,
        out_type=jax.ShapeDtypeStruct((B * T, D), jnp.float32),
        scratch_types=(
            [pltpu.VMEM((B * TCH,), jnp.int32),
             pltpu.VMEM((TCH, D), jnp.float32)]
            + [pltpu.VMEM((_PAIR * TCH, D), jnp.float32) for _ in range(_NBUF)]
            + [pltpu.SemaphoreType.DMA for _ in range(2 * _NBUF + 1)]
        ),
    )
    def body(idx_hbm, tok_hbm, pos_hbm, out_hbm, idx_v, pos_v, *bufs_sems):
        rows = bufs_sems[:_NBUF]
        gsem = bufs_sems[_NBUF:2 * _NBUF]
        wsem = bufs_sems[2 * _NBUF:3 * _NBUF]
        psem = bufs_sems[3 * _NBUF]
        wid = lax.axis_index("s") * NC + lax.axis_index("c")
        t0 = wid * TCH

        idx_d = [pltpu.async_copy(idx_hbm.at[b, pl.ds(t0, TCH)],
                                  idx_v.at[pl.ds(b * TCH, TCH)], psem)
                 for b in range(B)]
        pos_d = pltpu.async_copy(pos_hbm.at[pl.ds(t0, TCH)], pos_v, psem)
        for d in idx_d:
            d.wait()

        def gather(p):
            return pltpu.async_copy(
                tok_hbm.at[idx_v.at[pl.ds(p * _PAIR * TCH, _PAIR * TCH)]],
                rows[p % _NBUF], gsem[p % _NBUF])

        gd = [None] * _NBUF
        wd = [None] * _NBUF
        gd[0] = gather(0)
        pos_d.wait()

        SPLIT = 8
        for p in range(NP):
            cur = p % _NBUF
            gd[cur].wait()
            rcur = rows[cur]

            def per_row(r, carry, rcur=rcur):
                for j in range(D // _L):
                    sl = pl.ds(j * _L, _L)
                    pv = pos_v[r, sl]
                    for h in range(_PAIR):
                        plsc.addupdate(rcur.at[r + h * TCH, sl], pv)
                return carry

            lax.fori_loop(0, SPLIT, per_row, 0)
            if p + 1 < NP:
                nb = (p + 1) % _NBUF
                for d in (wd[nb] or ()):
                    d.wait()
                gd[nb] = gather(p + 1)
            lax.fori_loop(SPLIT, TCH, per_row, 0)
            wd[cur] = [
                pltpu.async_copy(
                    rcur.at[pl.ds(h * TCH, TCH)],
                    out_hbm.at[pl.ds((p * _PAIR + h) * T + t0, TCH)],
                    wsem[cur])
                for h in range(_PAIR)
            ]
        for ds_ in wd:
            for d in (ds_ or ()):
                d.wait()

    return body


def kernel(idx, tok_table, pos_table):
    B, T = idx.shape
    V, D = tok_table.shape
    info = plsc.get_sparse_core_info()
    NC, NS = info.num_cores, info.num_subcores
    fn = _emb_kernel(B, T, V, D, NC, NS)
    out = fn(idx.astype(jnp.int32), tok_table, pos_table)
    return out.reshape(B, T, D)


# speedup vs baseline: 1.1373x; 1.1110x over previous
import functools

import jax
import jax.numpy as jnp
from jax import lax
from jax.experimental import pallas as pl
from jax.experimental.pallas import tpu as pltpu
from jax.experimental.pallas import tpu_sc as plsc

_L = 16
_NBUF = 4


def _emb_kernel(B, T, V, D, NC, NS):
    NW = NC * NS
    TCH = T // NW
    mesh = plsc.VectorSubcoreMesh(core_axis_name="c", subcore_axis_name="s")

    @functools.partial(
        pl.kernel,
        mesh=mesh,
        out_type=jax.ShapeDtypeStruct((B * T, D), jnp.float32),
        scratch_types=(
            [pltpu.VMEM((B * TCH,), jnp.int32),
             pltpu.VMEM((TCH, D), jnp.float32)]
            + [pltpu.VMEM((TCH, D), jnp.float32) for _ in range(_NBUF)]
            + [pltpu.SemaphoreType.DMA for _ in range(2 * _NBUF + 1)]
        ),
    )
    def body(idx_hbm, tok_hbm, pos_hbm, out_hbm, idx_v, pos_v, *bufs_sems):
        rows = bufs_sems[:_NBUF]
        gsem = bufs_sems[_NBUF:2 * _NBUF]
        wsem = bufs_sems[2 * _NBUF:3 * _NBUF]
        psem = bufs_sems[3 * _NBUF]
        wid = lax.axis_index("s") * NC + lax.axis_index("c")
        t0 = wid * TCH

        idx_d = [pltpu.async_copy(idx_hbm.at[b, pl.ds(t0, TCH)],
                                  idx_v.at[pl.ds(b * TCH, TCH)], psem)
                 for b in range(B)]
        pos_d = pltpu.async_copy(pos_hbm.at[pl.ds(t0, TCH)], pos_v, psem)
        for d in idx_d:
            d.wait()

        def gather(b):
            return pltpu.async_copy(
                tok_hbm.at[idx_v.at[pl.ds(b * TCH, TCH)]],
                rows[b % _NBUF], gsem[b % _NBUF])

        gd = [None] * _NBUF
        wd = [None] * _NBUF
        gd[0] = gather(0)
        gd[1] = gather(1)
        pos_d.wait()

        SPLIT = 8
        for q in range(B // 2):
            b0, b1 = 2 * q, 2 * q + 1
            c0, c1 = b0 % _NBUF, b1 % _NBUF
            gd[c0].wait()
            gd[c1].wait()
            r0, r1 = rows[c0], rows[c1]

            def per_row(r, carry, r0=r0, r1=r1):
                for j in range(D // _L):
                    sl = pl.ds(j * _L, _L)
                    pv = pos_v[r, sl]
                    plsc.addupdate(r0.at[r, sl], pv)
                    plsc.addupdate(r1.at[r, sl], pv)
                return carry

            lax.fori_loop(0, SPLIT, per_row, 0)
            for b in (b0 + 2, b1 + 2):
                if b < B:
                    nb = b % _NBUF
                    if wd[nb] is not None:
                        wd[nb].wait()
                    gd[nb] = gather(b)
            lax.fori_loop(SPLIT, TCH, per_row, 0)
            wd[c0] = pltpu.async_copy(
                r0, out_hbm.at[pl.ds(b0 * T + t0, TCH)], wsem[c0])
            wd[c1] = pltpu.async_copy(
                r1, out_hbm.at[pl.ds(b1 * T + t0, TCH)], wsem[c1])
        for d in wd:
            if d is not None:
                d.wait()

    return body


def kernel(idx, tok_table, pos_table):
    B, T = idx.shape
    V, D = tok_table.shape
    info = plsc.get_sparse_core_info()
    NC, NS = info.num_cores, info.num_subcores
    fn = _emb_kernel(B, T, V, D, NC, NS)
    out = fn(idx.astype(jnp.int32), tok_table, pos_table)
    return out.reshape(B, T, D)


# speedup vs baseline: 1.1663x; 1.0255x over previous
import functools

import jax
import jax.numpy as jnp
from jax import lax
from jax.experimental import pallas as pl
from jax.experimental.pallas import tpu as pltpu
from jax.experimental.pallas import tpu_sc as plsc

_L = 16
_NBUF = 4


def _emb_kernel(B, T, V, D, NC, NS):
    NW = NC * NS
    TCH = T // NW
    mesh = plsc.VectorSubcoreMesh(core_axis_name="c", subcore_axis_name="s")

    @functools.partial(
        pl.kernel,
        mesh=mesh,
        out_type=jax.ShapeDtypeStruct((B * T, D), jnp.float32),
        scratch_types=(
            [pltpu.VMEM((B * TCH,), jnp.int32),
             pltpu.VMEM((TCH, D), jnp.float32)]
            + [pltpu.VMEM((TCH, D), jnp.float32) for _ in range(_NBUF)]
            + [pltpu.SemaphoreType.DMA for _ in range(2 * _NBUF + 1)]
        ),
    )
    def body(idx_hbm, tok_hbm, pos_hbm, out_hbm, idx_v, pos_v, *bufs_sems):
        rows = bufs_sems[:_NBUF]
        gsem = bufs_sems[_NBUF:2 * _NBUF]
        wsem = bufs_sems[2 * _NBUF:3 * _NBUF]
        psem = bufs_sems[3 * _NBUF]
        wid = lax.axis_index("s") * NC + lax.axis_index("c")
        t0 = wid * TCH

        idx_d = [pltpu.async_copy(idx_hbm.at[b, pl.ds(t0, TCH)],
                                  idx_v.at[pl.ds(b * TCH, TCH)], psem)
                 for b in range(B)]
        pos_d = pltpu.async_copy(pos_hbm.at[pl.ds(t0, TCH)], pos_v, psem)
        for d in idx_d:
            d.wait()

        def gather(b):
            return pltpu.async_copy(
                tok_hbm.at[idx_v.at[pl.ds(b * TCH, TCH)]],
                rows[b % _NBUF], gsem[b % _NBUF])

        gd = [None] * _NBUF
        wd = [None] * _NBUF
        gd[0] = gather(0)
        gd[1] = gather(1)
        pos_d.wait()

        SPLIT = 8
        for q in range(B // 2):
            b0, b1 = 2 * q, 2 * q + 1
            c0, c1 = b0 % _NBUF, b1 % _NBUF
            gd[c0].wait()
            gd[c1].wait()
            r0, r1 = rows[c0], rows[c1]

            def per_row(r, carry, r0=r0, r1=r1):
                for j in range(D // _L):
                    sl = pl.ds(j * _L, _L)
                    pv = pos_v[r, sl]
                    plsc.addupdate(r0.at[r, sl], pv)
                    plsc.addupdate(r1.at[r, sl], pv)
                return carry

            for b in (b0 + 2, b1 + 2):
                if b < B:
                    nb = b % _NBUF
                    if wd[nb] is not None:
                        wd[nb].wait()
                    gd[nb] = gather(b)
            lax.fori_loop(0, TCH, per_row, 0)
            wd[c0] = pltpu.async_copy(
                r0, out_hbm.at[pl.ds(b0 * T + t0, TCH)], wsem[c0])
            wd[c1] = pltpu.async_copy(
                r1, out_hbm.at[pl.ds(b1 * T + t0, TCH)], wsem[c1])
        for d in wd:
            if d is not None:
                d.wait()

    return body


def kernel(idx, tok_table, pos_table):
    B, T = idx.shape
    V, D = tok_table.shape
    info = plsc.get_sparse_core_info()
    NC, NS = info.num_cores, info.num_subcores
    fn = _emb_kernel(B, T, V, D, NC, NS)
    out = fn(idx.astype(jnp.int32), tok_table, pos_table)
    return out.reshape(B, T, D)


# speedup vs baseline: 1.2012x; 1.0299x over previous
import functools

import jax
import jax.numpy as jnp
from jax import lax
from jax.experimental import pallas as pl
from jax.experimental.pallas import tpu as pltpu
from jax.experimental.pallas import tpu_sc as plsc

_L = 16
_NBUF = 4


def _emb_kernel(B, T, V, D, NC, NS):
    NW = NC * NS
    TCH = T // NW
    mesh = plsc.VectorSubcoreMesh(core_axis_name="c", subcore_axis_name="s")

    @functools.partial(
        pl.kernel,
        mesh=mesh,
        out_type=jax.ShapeDtypeStruct((B * T, D), jnp.float32),
        scratch_types=(
            [pltpu.VMEM((B * TCH,), jnp.int32),
             pltpu.VMEM((TCH, D), jnp.float32)]
            + [pltpu.VMEM((TCH, D), jnp.float32) for _ in range(_NBUF)]
            + [pltpu.SemaphoreType.DMA for _ in range(2 * _NBUF + 3)]
        ),
    )
    def body(idx_hbm, tok_hbm, pos_hbm, out_hbm, idx_v, pos_v, *bufs_sems):
        rows = bufs_sems[:_NBUF]
        gsem = bufs_sems[_NBUF:2 * _NBUF]
        wsem = bufs_sems[2 * _NBUF:3 * _NBUF]
        psem = bufs_sems[3 * _NBUF]
        sA = bufs_sems[3 * _NBUF + 1]
        sB = bufs_sems[3 * _NBUF + 2]
        wid = lax.axis_index("s") * NC + lax.axis_index("c")
        t0 = wid * TCH

        def idx_copy(b, sem):
            return pltpu.async_copy(idx_hbm.at[b, pl.ds(t0, TCH)],
                                    idx_v.at[pl.ds(b * TCH, TCH)], sem)

        idx_d01 = [idx_copy(0, sA), idx_copy(1, sB)]
        idx_d = [idx_copy(b, psem) for b in range(2, B)]
        pos_d = pltpu.async_copy(pos_hbm.at[pl.ds(t0, TCH)], pos_v, psem)

        def gather(b):
            return pltpu.async_copy(
                tok_hbm.at[idx_v.at[pl.ds(b * TCH, TCH)]],
                rows[b % _NBUF], gsem[b % _NBUF])

        gd = [None] * _NBUF
        wd = [None] * _NBUF
        idx_d01[0].wait()
        gd[0] = gather(0)
        idx_d01[1].wait()
        gd[1] = gather(1)
        for d in idx_d:
            d.wait()
        pos_d.wait()

        SPLIT = 8
        for q in range(B // 2):
            b0, b1 = 2 * q, 2 * q + 1
            c0, c1 = b0 % _NBUF, b1 % _NBUF
            gd[c0].wait()
            gd[c1].wait()
            r0, r1 = rows[c0], rows[c1]

            def per_row(i, carry, r0=r0, r1=r1):
                for u in range(2):
                    r = 2 * i + u
                    for j in range(D // _L):
                        sl = pl.ds(j * _L, _L)
                        pv = pos_v[r, sl]
                        plsc.addupdate(r0.at[r, sl], pv)
                        plsc.addupdate(r1.at[r, sl], pv)
                return carry

            for b in (b0 + 2, b1 + 2):
                if b < B:
                    nb = b % _NBUF
                    if wd[nb] is not None:
                        wd[nb].wait()
                    gd[nb] = gather(b)
            lax.fori_loop(0, TCH // 2, per_row, 0)
            wd[c0] = pltpu.async_copy(
                r0, out_hbm.at[pl.ds(b0 * T + t0, TCH)], wsem[c0])
            wd[c1] = pltpu.async_copy(
                r1, out_hbm.at[pl.ds(b1 * T + t0, TCH)], wsem[c1])
        for d in wd:
            if d is not None:
                d.wait()

    return body


def kernel(idx, tok_table, pos_table):
    B, T = idx.shape
    V, D = tok_table.shape
    info = plsc.get_sparse_core_info()
    NC, NS = info.num_cores, info.num_subcores
    fn = _emb_kernel(B, T, V, D, NC, NS)
    out = fn(idx.astype(jnp.int32), tok_table, pos_table)
    return out.reshape(B, T, D)


# speedup vs baseline: 1.2077x; 1.0054x over previous
import functools

import jax
import jax.numpy as jnp
from jax import lax
from jax.experimental import pallas as pl
from jax.experimental.pallas import tpu as pltpu
from jax.experimental.pallas import tpu_sc as plsc

_L = 16
_NBUF = 4


def _emb_kernel(B, T, V, D, NC, NS):
    NW = NC * NS
    TCH = T // NW
    mesh = plsc.VectorSubcoreMesh(core_axis_name="c", subcore_axis_name="s")

    @functools.partial(
        pl.kernel,
        mesh=mesh,
        out_type=jax.ShapeDtypeStruct((B * T, D), jnp.float32),
        scratch_types=(
            [pltpu.VMEM((B * TCH,), jnp.int32),
             pltpu.VMEM((TCH, D), jnp.float32)]
            + [pltpu.VMEM((TCH, D), jnp.float32) for _ in range(_NBUF)]
            + [pltpu.SemaphoreType.DMA for _ in range(2 * _NBUF + 3)]
        ),
    )
    def body(idx_hbm, tok_hbm, pos_hbm, out_hbm, idx_v, pos_v, *bufs_sems):
        rows = bufs_sems[:_NBUF]
        gsem = bufs_sems[_NBUF:2 * _NBUF]
        wsem = bufs_sems[2 * _NBUF:3 * _NBUF]
        psem = bufs_sems[3 * _NBUF]
        sA = bufs_sems[3 * _NBUF + 1]
        sB = bufs_sems[3 * _NBUF + 2]
        wid = lax.axis_index("c") * NS + lax.axis_index("s")
        t0 = wid * TCH

        def idx_copy(b, sem):
            return pltpu.async_copy(idx_hbm.at[b, pl.ds(t0, TCH)],
                                    idx_v.at[pl.ds(b * TCH, TCH)], sem)

        idx_d01 = [idx_copy(0, sA), idx_copy(1, sB)]
        idx_d = [idx_copy(b, psem) for b in range(2, B)]
        pos_d = pltpu.async_copy(pos_hbm.at[pl.ds(t0, TCH)], pos_v, psem)

        def gather(b):
            return pltpu.async_copy(
                tok_hbm.at[idx_v.at[pl.ds(b * TCH, TCH)]],
                rows[b % _NBUF], gsem[b % _NBUF])

        gd = [None] * _NBUF
        wd = [None] * _NBUF
        idx_d01[0].wait()
        gd[0] = gather(0)
        idx_d01[1].wait()
        gd[1] = gather(1)
        for d in idx_d:
            d.wait()
        pos_d.wait()

        SPLIT = 8
        for q in range(B // 2):
            b0, b1 = 2 * q, 2 * q + 1
            c0, c1 = b0 % _NBUF, b1 % _NBUF
            gd[c0].wait()
            gd[c1].wait()
            r0, r1 = rows[c0], rows[c1]

            def per_row(i, carry, r0=r0, r1=r1):
                for u in range(2):
                    r = 2 * i + u
                    for j in range(D // _L):
                        sl = pl.ds(j * _L, _L)
                        pv = pos_v[r, sl]
                        plsc.addupdate(r0.at[r, sl], pv)
                        plsc.addupdate(r1.at[r, sl], pv)
                return carry

            for b in (b0 + 2, b1 + 2):
                if b < B:
                    nb = b % _NBUF
                    if wd[nb] is not None:
                        wd[nb].wait()
                    gd[nb] = gather(b)
            lax.fori_loop(0, TCH // 2, per_row, 0)
            wd[c0] = pltpu.async_copy(
                r0, out_hbm.at[pl.ds(b0 * T + t0, TCH)], wsem[c0])
            wd[c1] = pltpu.async_copy(
                r1, out_hbm.at[pl.ds(b1 * T + t0, TCH)], wsem[c1])
        for d in wd:
            if d is not None:
                d.wait()

    return body


def kernel(idx, tok_table, pos_table):
    B, T = idx.shape
    V, D = tok_table.shape
    info = plsc.get_sparse_core_info()
    NC, NS = info.num_cores, info.num_subcores
    fn = _emb_kernel(B, T, V, D, NC, NS)
    out = fn(idx.astype(jnp.int32), tok_table, pos_table)
    return out.reshape(B, T, D)


# speedup vs baseline: 1.2107x; 1.0024x over previous
import functools

import jax
import jax.numpy as jnp
from jax import lax
from jax.experimental import pallas as pl
from jax.experimental.pallas import tpu as pltpu
from jax.experimental.pallas import tpu_sc as plsc

_L = 16
_NBUF = 4


def _emb_kernel(B, T, V, D, NC, NS):
    NW = NC * NS
    TCH = T // NW
    mesh = plsc.VectorSubcoreMesh(core_axis_name="c", subcore_axis_name="s")

    @functools.partial(
        pl.kernel,
        mesh=mesh,
        out_type=jax.ShapeDtypeStruct((B * T, D), jnp.float32),
        scratch_types=(
            [pltpu.VMEM((B * TCH,), jnp.int32),
             pltpu.VMEM((TCH, D), jnp.float32)]
            + [pltpu.VMEM((TCH, D), jnp.float32) for _ in range(_NBUF)]
            + [pltpu.SemaphoreType.DMA for _ in range(2 * _NBUF + 3)]
        ),
    )
    def body(idx_hbm, tok_hbm, pos_hbm, out_hbm, idx_v, pos_v, *bufs_sems):
        rows = bufs_sems[:_NBUF]
        gsem = bufs_sems[_NBUF:2 * _NBUF]
        wsem = bufs_sems[2 * _NBUF:3 * _NBUF]
        psem = bufs_sems[3 * _NBUF]
        sA = bufs_sems[3 * _NBUF + 1]
        sB = bufs_sems[3 * _NBUF + 2]
        wid = lax.axis_index("c") * NS + lax.axis_index("s")
        t0 = wid * TCH

        def idx_copy(b, sem):
            return pltpu.async_copy(idx_hbm.at[b, pl.ds(t0, TCH)],
                                    idx_v.at[pl.ds(b * TCH, TCH)], sem)

        idx_d01 = [idx_copy(0, sA), idx_copy(1, sB)]
        idx_d = [idx_copy(b, psem) for b in range(2, B)]
        pos_d = pltpu.async_copy(pos_hbm.at[pl.ds(t0, TCH)], pos_v, psem)

        def gather(b):
            return pltpu.async_copy(
                tok_hbm.at[idx_v.at[pl.ds(b * TCH, TCH)]],
                rows[b % _NBUF], gsem[b % _NBUF])

        gd = [None] * _NBUF
        wd = [None] * _NBUF
        idx_d01[0].wait()
        gd[0] = gather(0)
        idx_d01[1].wait()
        gd[1] = gather(1)
        for d in idx_d:
            d.wait()
        pos_d.wait()

        SPLIT = 8
        for q in range(B // 2):
            b0, b1 = 2 * q, 2 * q + 1
            c0, c1 = b0 % _NBUF, b1 % _NBUF
            gd[c0].wait()
            gd[c1].wait()
            r0, r1 = rows[c0], rows[c1]

            def per_row(i, carry, r0=r0, r1=r1):
                for u in range(2):
                    r = 2 * i + u
                    for j in range(D // _L):
                        sl = pl.ds(j * _L, _L)
                        pv = pos_v[r, sl]
                        plsc.addupdate(r0.at[r, sl], pv)
                        plsc.addupdate(r1.at[r, sl], pv)
                return carry

            for b in (b0 + 2, b1 + 2):
                if b < B:
                    nb = b % _NBUF
                    if wd[nb] is not None:
                        wd[nb].wait()
                    gd[nb] = gather(b)
            if q + 1 < B // 2:
                lax.fori_loop(0, TCH // 2, per_row, 0)
                wd[c0] = pltpu.async_copy(
                    r0, out_hbm.at[pl.ds(b0 * T + t0, TCH)], wsem[c0])
                wd[c1] = pltpu.async_copy(
                    r1, out_hbm.at[pl.ds(b1 * T + t0, TCH)], wsem[c1])
            else:
                H = TCH // 2
                lax.fori_loop(0, H // 2, per_row, 0)
                pltpu.async_copy(
                    r0.at[pl.ds(0, H)],
                    out_hbm.at[pl.ds(b0 * T + t0, H)], wsem[c0])
                pltpu.async_copy(
                    r1.at[pl.ds(0, H)],
                    out_hbm.at[pl.ds(b1 * T + t0, H)], wsem[c1])
                lax.fori_loop(H // 2, TCH // 2, per_row, 0)
                wd[c0] = pltpu.async_copy(
                    r0.at[pl.ds(H, H)],
                    out_hbm.at[pl.ds(b0 * T + t0 + H, H)], wsem[c0])
                wd[c1] = pltpu.async_copy(
                    r1.at[pl.ds(H, H)],
                    out_hbm.at[pl.ds(b1 * T + t0 + H, H)], wsem[c1])
                pltpu.make_async_copy(
                    r0.at[pl.ds(0, H)],
                    out_hbm.at[pl.ds(b0 * T + t0, H)], wsem[c0]).wait()
                pltpu.make_async_copy(
                    r1.at[pl.ds(0, H)],
                    out_hbm.at[pl.ds(b1 * T + t0, H)], wsem[c1]).wait()
        for d in wd:
            if d is not None:
                d.wait()

    return body


def kernel(idx, tok_table, pos_table):
    B, T = idx.shape
    V, D = tok_table.shape
    info = plsc.get_sparse_core_info()
    NC, NS = info.num_cores, info.num_subcores
    fn = _emb_kernel(B, T, V, D, NC, NS)
    out = fn(idx.astype(jnp.int32), tok_table, pos_table)
    return out.reshape(B, T, D)


# speedup vs baseline: 1.2287x; 1.0149x over previous
import functools

import jax
import jax.numpy as jnp
from jax import lax
from jax.experimental import pallas as pl
from jax.experimental.pallas import tpu as pltpu
from jax.experimental.pallas import tpu_sc as plsc

_L = 16
_NBUF = 4


def _emb_kernel(B, T, V, D, NC, NS):
    NW = NC * NS
    TCH = T // NW
    mesh = plsc.VectorSubcoreMesh(core_axis_name="c", subcore_axis_name="s")

    @functools.partial(
        pl.kernel,
        mesh=mesh,
        out_type=jax.ShapeDtypeStruct((B * T, D), jnp.float32),
        scratch_types=(
            [pltpu.VMEM((B * TCH,), jnp.int32),
             pltpu.VMEM((TCH, D), jnp.float32)]
            + [pltpu.VMEM((TCH, D), jnp.float32) for _ in range(_NBUF)]
            + [pltpu.SemaphoreType.DMA for _ in range(2 * _NBUF + 3)]
        ),
    )
    def body(idx_hbm, tok_hbm, pos_hbm, out_hbm, idx_v, pos_v, *bufs_sems):
        rows = bufs_sems[:_NBUF]
        gsem = bufs_sems[_NBUF:2 * _NBUF]
        wsem = bufs_sems[2 * _NBUF:3 * _NBUF]
        psem = bufs_sems[3 * _NBUF]
        sA = bufs_sems[3 * _NBUF + 1]
        sB = bufs_sems[3 * _NBUF + 2]
        wid = lax.axis_index("c") * NS + lax.axis_index("s")
        t0 = wid * TCH

        def idx_copy(b, sem):
            return pltpu.async_copy(idx_hbm.at[b, pl.ds(t0, TCH)],
                                    idx_v.at[pl.ds(b * TCH, TCH)], sem)

        idx_d01 = [idx_copy(0, sA), idx_copy(1, sB)]
        idx_d = [idx_copy(b, psem) for b in range(2, B)]
        pos_d = pltpu.async_copy(pos_hbm.at[pl.ds(t0, TCH)], pos_v, psem)

        def gather(b):
            return pltpu.async_copy(
                tok_hbm.at[idx_v.at[pl.ds(b * TCH, TCH)]],
                rows[b % _NBUF], gsem[b % _NBUF])

        gd = [None] * _NBUF
        wd = [None] * _NBUF
        idx_d01[0].wait()
        gd[0] = gather(0)
        idx_d01[1].wait()
        gd[1] = gather(1)
        for d in idx_d:
            d.wait()
        pos_d.wait()

        for q in range(B // 2):
            b0, b1 = 2 * q, 2 * q + 1
            c0, c1 = b0 % _NBUF, b1 % _NBUF
            gd[c0].wait()
            gd[c1].wait()
            r0, r1 = rows[c0], rows[c1]

            def per_row(i, carry, r0=r0, r1=r1):
                for u in range(2):
                    r = 2 * i + u
                    for j in range(D // _L):
                        sl = pl.ds(j * _L, _L)
                        pv = pos_v[r, sl]
                        plsc.addupdate(r0.at[r, sl], pv)
                        plsc.addupdate(r1.at[r, sl], pv)
                return carry

            for b in (b0 + 2, b1 + 2):
                if b < B:
                    nb = b % _NBUF
                    if wd[nb] is not None:
                        wd[nb].wait()
                    gd[nb] = gather(b)
            if q + 1 < B // 2:
                lax.fori_loop(0, TCH // 2, per_row, 0)
                wd[c0] = pltpu.async_copy(
                    r0, out_hbm.at[pl.ds(b0 * T + t0, TCH)], wsem[c0])
                wd[c1] = pltpu.async_copy(
                    r1, out_hbm.at[pl.ds(b1 * T + t0, TCH)], wsem[c1])
            else:
                H = TCH // 2
                lax.fori_loop(0, H // 2, per_row, 0)
                pltpu.async_copy(
                    r0.at[pl.ds(0, H)],
                    out_hbm.at[pl.ds(b0 * T + t0, H)], wsem[c0])
                pltpu.async_copy(
                    r1.at[pl.ds(0, H)],
                    out_hbm.at[pl.ds(b1 * T + t0, H)], wsem[c1])
                lax.fori_loop(H // 2, TCH // 2, per_row, 0)
                wd[c0] = pltpu.async_copy(
                    r0.at[pl.ds(H, H)],
                    out_hbm.at[pl.ds(b0 * T + t0 + H, H)], wsem[c0])
                wd[c1] = pltpu.async_copy(
                    r1.at[pl.ds(H, H)],
                    out_hbm.at[pl.ds(b1 * T + t0 + H, H)], wsem[c1])
                pltpu.make_async_copy(
                    r0.at[pl.ds(0, H)],
                    out_hbm.at[pl.ds(b0 * T + t0, H)], wsem[c0]).wait()
                pltpu.make_async_copy(
                    r1.at[pl.ds(0, H)],
                    out_hbm.at[pl.ds(b1 * T + t0, H)], wsem[c1]).wait()
        for d in wd:
            if d is not None:
                d.wait()

    return body


def kernel(idx, tok_table, pos_table):
    B, T = idx.shape
    V, D = tok_table.shape
    info = plsc.get_sparse_core_info()
    NC, NS = info.num_cores, info.num_subcores
    fn = _emb_kernel(B, T, V, D, NC, NS)
    out = fn(idx.astype(jnp.int32), tok_table, pos_table)
    return out.reshape(B, T, D)
